# Initial kernel scaffold; baseline (speedup 1.0000x reference)
import functools

import jax
import jax.numpy as jnp
from jax import lax
from jax.experimental import pallas as pl
from jax.experimental.pallas import tpu as pltpu
from jax.experimental.pallas import tpu_sc as plsc

N = 10000
NP = 10240
E = 320000
F1 = 128
F2 = 16
NC = 2
NS = 16
NW = NC * NS
EPW = E // NW
K = 80
NCHUNK = EPW // K
RPT = NP // NS

_MESH = plsc.VectorSubcoreMesh(
    core_axis_name="c", subcore_axis_name="s", num_cores=NC, num_subcores=NS)


@functools.partial(
    pl.kernel,
    out_type=jax.ShapeDtypeStruct((NC, NP), jnp.float32),
    mesh=_MESH,
    scratch_types=[
        pltpu.VMEM((K,), jnp.int32),
        pltpu.VMEM((K,), jnp.float32),
        pltpu.VMEM_SHARED((NP,), jnp.float32),
    ],
)
def _deg_sc(edge_ref, zeros_ref, ones_ref, out_ref, idx_v, ones_v, acc):
    c = lax.axis_index("c")
    s = lax.axis_index("s")
    wid = c * NS + s
    pltpu.sync_copy(ones_ref, ones_v)
    pltpu.sync_copy(zeros_ref.at[pl.ds(s * RPT, RPT)],
                    acc.at[pl.ds(s * RPT, RPT)])
    plsc.subcore_barrier()

    def body(i, carry):
        base = wid * EPW + i * K
        pltpu.sync_copy(edge_ref.at[1, pl.ds(base, K)], idx_v)
        pltpu.sync_copy(ones_v, acc.at[idx_v], add=True)
        return carry

    lax.fori_loop(0, NCHUNK, body, 0)
    plsc.subcore_barrier()
    pltpu.sync_copy(acc.at[pl.ds(s * RPT, RPT)],
                    out_ref.at[c, pl.ds(s * RPT, RPT)])


def _make_row_scatter(D):
    @functools.partial(
        pl.kernel,
        out_type=jax.ShapeDtypeStruct((NC, NP, D), jnp.float32),
        mesh=_MESH,
        scratch_types=[
            pltpu.VMEM((K,), jnp.int32),
            pltpu.VMEM((K,), jnp.int32),
            pltpu.VMEM((K, D), jnp.float32),
            pltpu.SemaphoreType.DMA,
            pltpu.VMEM_SHARED((NP, D), jnp.float32),
        ],
    )
    def row_scatter(hp_ref, edge_ref, zeros_ref, out_ref,
                    sidx, didx, rows, sem, acc):
        c = lax.axis_index("c")
        s = lax.axis_index("s")
        wid = c * NS + s
        pltpu.sync_copy(zeros_ref.at[pl.ds(s * RPT, RPT)],
                        acc.at[pl.ds(s * RPT, RPT)])
        plsc.subcore_barrier()

        def body(i, carry):
            base = wid * EPW + i * K
            pltpu.sync_copy(edge_ref.at[0, pl.ds(base, K)], sidx)
            pltpu.sync_copy(edge_ref.at[1, pl.ds(base, K)], didx)
            pltpu.async_copy(hp_ref.at[sidx], rows, sem).wait()
            pltpu.sync_copy(rows, acc.at[didx], add=True)
            return carry

        lax.fori_loop(0, NCHUNK, body, 0)
        plsc.subcore_barrier()
        pltpu.sync_copy(acc.at[pl.ds(s * RPT, RPT)],
                        out_ref.at[c, pl.ds(s * RPT, RPT)])

    return row_scatter


_scatter128 = _make_row_scatter(F1)
_scatter16 = _make_row_scatter(F2)


_RB = 2048


def _stage1_body(deg_ref, x_ref, w1_ref, dinv_ref, h1p_ref):
    deg = deg_ref[0, :] + deg_ref[1, :] + 1.0
    dinv = lax.rsqrt(deg)
    h = jnp.dot(x_ref[...], w1_ref[...], preferred_element_type=jnp.float32)
    dinv_ref[...] = dinv
    h1p_ref[...] = h * dinv[:, None]


def _stage1(deg2, x_pad, W1):
    return pl.pallas_call(
        _stage1_body,
        grid=(NP // _RB,),
        in_specs=[
            pl.BlockSpec((2, _RB), lambda i: (0, i)),
            pl.BlockSpec((_RB, F1), lambda i: (i, 0)),
            pl.BlockSpec((F1, F1), lambda i: (0, 0)),
        ],
        out_specs=[
            pl.BlockSpec((_RB,), lambda i: (i,)),
            pl.BlockSpec((_RB, F1), lambda i: (i, 0)),
        ],
        out_shape=[
            jax.ShapeDtypeStruct((NP,), jnp.float32),
            jax.ShapeDtypeStruct((NP, F1), jnp.float32),
        ],
    )(deg2, x_pad, W1)


def _stage2_body(s1_ref, h1p_ref, dinv_ref, b1_ref, w2_ref, h2p_ref):
    t = s1_ref[0] + s1_ref[1] + h1p_ref[...]
    dinv = dinv_ref[...]
    out1 = jnp.maximum(dinv[:, None] * t + b1_ref[...][None, :], 0.0)
    h2 = jnp.dot(out1, w2_ref[...], preferred_element_type=jnp.float32)
    h2p_ref[...] = h2 * dinv[:, None]


def _stage2(s1, h1p, dinv, b1, W2):
    return pl.pallas_call(
        _stage2_body,
        grid=(NP // _RB,),
        in_specs=[
            pl.BlockSpec((2, _RB, F1), lambda i: (0, i, 0)),
            pl.BlockSpec((_RB, F1), lambda i: (i, 0)),
            pl.BlockSpec((_RB,), lambda i: (i,)),
            pl.BlockSpec((F1,), lambda i: (0,)),
            pl.BlockSpec((F1, F2), lambda i: (0, 0)),
        ],
        out_specs=pl.BlockSpec((_RB, F2), lambda i: (i, 0)),
        out_shape=jax.ShapeDtypeStruct((NP, F2), jnp.float32),
    )(s1, h1p, dinv, b1, W2)


def _stage3_body(s2_ref, h2p_ref, dinv_ref, b2_ref, out_ref):
    t = s2_ref[0] + s2_ref[1] + h2p_ref[...]
    o = dinv_ref[...][:, None] * t + b2_ref[...][None, :]
    m = jnp.max(o, axis=1, keepdims=True)
    lse = m + jnp.log(jnp.sum(jnp.exp(o - m), axis=1, keepdims=True))
    out_ref[...] = o - lse


def _stage3(s2, h2p, dinv, b2):
    return pl.pallas_call(
        _stage3_body,
        grid=(NP // _RB,),
        in_specs=[
            pl.BlockSpec((2, _RB, F2), lambda i: (0, i, 0)),
            pl.BlockSpec((_RB, F2), lambda i: (i, 0)),
            pl.BlockSpec((_RB,), lambda i: (i,)),
            pl.BlockSpec((F2,), lambda i: (0,)),
        ],
        out_specs=pl.BlockSpec((_RB, F2), lambda i: (i, 0)),
        out_shape=jax.ShapeDtypeStruct((NP, F2), jnp.float32),
    )(s2, h2p, dinv, b2)


def kernel(x, edge_index, W1, b1, W2, b2):
    ei = edge_index.astype(jnp.int32)
    x_pad = jnp.zeros((NP, F1), jnp.float32).at[:N, :].set(x)
    zc = jnp.zeros((NP,), jnp.float32)
    ones = jnp.ones((K,), jnp.float32)
    z1 = jnp.zeros((NP, F1), jnp.float32)
    z2 = jnp.zeros((NP, F2), jnp.float32)

    deg2 = _deg_sc(ei, zc, ones)
    dinv, h1p = _stage1(deg2, x_pad, W1)
    s1 = _scatter128(h1p, ei, z1)
    h2p = _stage2(s1, h1p, dinv, b1, W2)
    s2 = _scatter16(h2p, ei, z2)
    out = _stage3(s2, h2p, dinv, b2)
    return out[:N, :]


# speedup vs baseline: 13.3215x; 13.3215x over previous
import functools

import jax
import jax.numpy as jnp
from jax import lax
from jax.experimental import pallas as pl
from jax.experimental.pallas import tpu as pltpu
from jax.experimental.pallas import tpu_sc as plsc

N = 10000
NP = 10240
E = 320000
F1 = 128
F2 = 16
NC = 2
NS = 16
NW = NC * NS
EPW = E // NW
K = 80
NCHUNK = EPW // K
RPT = NP // NS

_MESH = plsc.VectorSubcoreMesh(
    core_axis_name="c", subcore_axis_name="s", num_cores=NC, num_subcores=NS)


@functools.partial(
    pl.kernel,
    out_type=jax.ShapeDtypeStruct((NC, NP), jnp.float32),
    mesh=_MESH,
    scratch_types=[
        pltpu.VMEM((K,), jnp.int32),
        pltpu.VMEM((K,), jnp.float32),
        pltpu.VMEM_SHARED((NP,), jnp.float32),
    ],
)
def _deg_sc(dst_ref, zeros_ref, ones_ref, out_ref, idx_v, ones_v, acc):
    c = lax.axis_index("c")
    s = lax.axis_index("s")
    wid = c * NS + s
    pltpu.sync_copy(ones_ref, ones_v)
    pltpu.sync_copy(zeros_ref.at[pl.ds(s * RPT, RPT)],
                    acc.at[pl.ds(s * RPT, RPT)])
    plsc.subcore_barrier()

    def body(i, carry):
        base = wid * EPW + i * K
        pltpu.sync_copy(dst_ref.at[pl.ds(base, K)], idx_v)
        pltpu.sync_copy(ones_v, acc.at[idx_v], add=True)
        return carry

    lax.fori_loop(0, NCHUNK, body, 0)
    plsc.subcore_barrier()
    pltpu.sync_copy(acc.at[pl.ds(s * RPT, RPT)],
                    out_ref.at[c, pl.ds(s * RPT, RPT)])


def _make_row_scatter(D):
    @functools.partial(
        pl.kernel,
        out_type=jax.ShapeDtypeStruct((NC, NP, D), jnp.float32),
        mesh=_MESH,
        scratch_types=[
            pltpu.VMEM((K,), jnp.int32),
            pltpu.VMEM((K,), jnp.int32),
            pltpu.VMEM((K, D), jnp.float32),
            pltpu.SemaphoreType.DMA,
            pltpu.VMEM_SHARED((NP, D), jnp.float32),
        ],
    )
    def row_scatter(hp_ref, src_ref, dst_ref, zeros_ref, out_ref,
                    sidx, didx, rows, sem, acc):
        c = lax.axis_index("c")
        s = lax.axis_index("s")
        wid = c * NS + s
        pltpu.sync_copy(zeros_ref.at[pl.ds(s * RPT, RPT)],
                        acc.at[pl.ds(s * RPT, RPT)])
        plsc.subcore_barrier()

        def body(i, carry):
            base = wid * EPW + i * K
            pltpu.sync_copy(src_ref.at[pl.ds(base, K)], sidx)
            pltpu.sync_copy(dst_ref.at[pl.ds(base, K)], didx)
            pltpu.async_copy(hp_ref.at[sidx], rows, sem).wait()
            pltpu.sync_copy(rows, acc.at[didx], add=True)
            return carry

        lax.fori_loop(0, NCHUNK, body, 0)
        plsc.subcore_barrier()
        pltpu.sync_copy(acc.at[pl.ds(s * RPT, RPT)],
                        out_ref.at[c, pl.ds(s * RPT, RPT)])

    return row_scatter


_scatter128 = _make_row_scatter(F1)


_RB = 2048


def _stage1_body(deg_ref, x_ref, w1_ref, dinv_ref, h1p_ref):
    deg = deg_ref[0, :] + deg_ref[1, :] + 1.0
    dinv = lax.rsqrt(deg)
    h = jnp.dot(x_ref[...], w1_ref[...], preferred_element_type=jnp.float32)
    dinv_ref[...] = dinv
    h1p_ref[...] = h * dinv[:, None]


def _stage1(deg2, x_pad, W1):
    return pl.pallas_call(
        _stage1_body,
        grid=(NP // _RB,),
        in_specs=[
            pl.BlockSpec((2, _RB), lambda i: (0, i)),
            pl.BlockSpec((_RB, F1), lambda i: (i, 0)),
            pl.BlockSpec((F1, F1), lambda i: (0, 0)),
        ],
        out_specs=[
            pl.BlockSpec((_RB,), lambda i: (i,)),
            pl.BlockSpec((_RB, F1), lambda i: (i, 0)),
        ],
        out_shape=[
            jax.ShapeDtypeStruct((NP,), jnp.float32),
            jax.ShapeDtypeStruct((NP, F1), jnp.float32),
        ],
    )(deg2, x_pad, W1)


def _stage2_body(s1_ref, h1p_ref, dinv_ref, b1_ref, g2_ref):
    t = s1_ref[0] + s1_ref[1] + h1p_ref[...]
    dinv = dinv_ref[...]
    out1 = jnp.maximum(dinv[:, None] * t + b1_ref[...][None, :], 0.0)
    g2_ref[...] = out1 * dinv[:, None]


def _stage2(s1, h1p, dinv, b1):
    return pl.pallas_call(
        _stage2_body,
        grid=(NP // _RB,),
        in_specs=[
            pl.BlockSpec((2, _RB, F1), lambda i: (0, i, 0)),
            pl.BlockSpec((_RB, F1), lambda i: (i, 0)),
            pl.BlockSpec((_RB,), lambda i: (i,)),
            pl.BlockSpec((F1,), lambda i: (0,)),
        ],
        out_specs=pl.BlockSpec((_RB, F1), lambda i: (i, 0)),
        out_shape=jax.ShapeDtypeStruct((NP, F1), jnp.float32),
    )(s1, h1p, dinv, b1)


def _stage3_body(s2_ref, g2_ref, dinv_ref, b2_ref, w2_ref, out_ref):
    t = s2_ref[0] + s2_ref[1] + g2_ref[...]
    h2 = jnp.dot(t, w2_ref[...], preferred_element_type=jnp.float32)
    o = dinv_ref[...][:, None] * h2 + b2_ref[...][None, :]
    m = jnp.max(o, axis=1, keepdims=True)
    lse = m + jnp.log(jnp.sum(jnp.exp(o - m), axis=1, keepdims=True))
    out_ref[...] = o - lse


def _stage3(s2, g2, dinv, b2, W2):
    return pl.pallas_call(
        _stage3_body,
        grid=(NP // _RB,),
        in_specs=[
            pl.BlockSpec((2, _RB, F1), lambda i: (0, i, 0)),
            pl.BlockSpec((_RB, F1), lambda i: (i, 0)),
            pl.BlockSpec((_RB,), lambda i: (i,)),
            pl.BlockSpec((F2,), lambda i: (0,)),
            pl.BlockSpec((F1, F2), lambda i: (0, 0)),
        ],
        out_specs=pl.BlockSpec((_RB, F2), lambda i: (i, 0)),
        out_shape=jax.ShapeDtypeStruct((NP, F2), jnp.float32),
    )(s2, g2, dinv, b2, W2)


def kernel(x, edge_index, W1, b1, W2, b2):
    ei = edge_index.astype(jnp.int32)
    src = ei[0]
    dst = ei[1]
    x_pad = jnp.zeros((NP, F1), jnp.float32).at[:N, :].set(x)
    zc = jnp.zeros((NP,), jnp.float32)
    ones = jnp.ones((K,), jnp.float32)
    z1 = jnp.zeros((NP, F1), jnp.float32)

    deg2 = _deg_sc(dst, zc, ones)
    dinv, h1p = _stage1(deg2, x_pad, W1)
    s1 = _scatter128(h1p, src, dst, z1)
    g2 = _stage2(s1, h1p, dinv, b1)
    s2 = _scatter128(g2, src, dst, z1)
    out = _stage3(s2, g2, dinv, b2, W2)
    return out[:N, :]
